# baseline (speedup 1.0000x reference)
import functools

import jax
import jax.numpy as jnp
from jax import lax
from jax.experimental import pallas as pl
from jax.experimental.pallas import tpu as pltpu
from jax.experimental.pallas import tpu_sc as plsc

N = 10000
E = 320000
D = 128
H1 = 64
H2 = 32
K = 16

M = 2 * N
EPT = E // 16
C = 80
NCHUNK = EPT // C
RPT = N // 16

_mesh = plsc.VectorSubcoreMesh(core_axis_name="c", subcore_axis_name="s")
_sc_params = pltpu.CompilerParams(use_tc_tiling_on_sc=False)


_NIB = 8
_IAH = 4
_SDEP = 4
_NBUF = 4
_GAH = 2


@functools.partial(
    pl.kernel,
    out_type=jax.ShapeDtypeStruct((M, 16), jnp.float32),
    mesh=_mesh,
    compiler_params=_sc_params,
    scratch_types=[
        pltpu.VMEM((_NIB, C), jnp.int32),
        pltpu.VMEM((C, 16), jnp.float32),
        pltpu.VMEM_SHARED((N, 16), jnp.float32),
        pltpu.SemaphoreType.DMA((_NIB,)),
        pltpu.SemaphoreType.DMA,
    ],
)
def _sc_degree(dst_cat, ones_rows, zeros16, out, didx, ones_v, acc, isem, dsem):
    c = lax.axis_index("c")
    s = lax.axis_index("s")
    ebase = c * E + s * EPT
    pltpu.sync_copy(ones_rows, ones_v)
    pltpu.sync_copy(zeros16.at[pl.ds(s * RPT, RPT)], acc.at[pl.ds(s * RPT, RPT)])
    plsc.subcore_barrier()

    def i_start(j):
        b = j % _NIB
        pltpu.async_copy(dst_cat.at[pl.ds(ebase + j * C, C)], didx.at[b],
                         isem.at[b])

    def i_wait(j):
        b = j % _NIB
        pltpu.make_async_copy(dst_cat.at[pl.ds(ebase + j * C, C)], didx.at[b],
                              isem.at[b]).wait()

    def s_drain(_j):
        pltpu.make_async_copy(ones_v, acc.at[didx.at[_j % _NIB]], dsem).wait()

    for t in range(_IAH):
        i_start(t)

    def body(j, carry):
        @pl.when(j >= _SDEP)
        def _():
            s_drain(j - _SDEP)

        @pl.when(j + _IAH < NCHUNK)
        def _():
            i_start(j + _IAH)

        i_wait(j)
        pltpu.async_copy(ones_v, acc.at[didx.at[j % _NIB]], dsem, add=True)
        return carry

    lax.fori_loop(0, NCHUNK, body, 0)
    for t in range(_SDEP):
        s_drain(NCHUNK - _SDEP + t)
    plsc.subcore_barrier()
    pltpu.sync_copy(acc.at[pl.ds(s * RPT, RPT)],
                    out.at[pl.ds(c * N + s * RPT, RPT)])


def _make_sc_scatter(H):
    @functools.partial(
        pl.kernel,
        out_type=jax.ShapeDtypeStruct((M, H), jnp.bfloat16),
        mesh=_mesh,
        compiler_params=_sc_params,
        scratch_types=[
            pltpu.VMEM((_NIB, C), jnp.int32),
            pltpu.VMEM((_NIB, C), jnp.int32),
            pltpu.VMEM((_NBUF, C, H), jnp.bfloat16),
            pltpu.VMEM_SHARED((N, H), jnp.bfloat16),
            pltpu.SemaphoreType.DMA((_NIB,)),
            pltpu.SemaphoreType.DMA((_NBUF,)),
            pltpu.SemaphoreType.DMA((_NBUF,)),
        ],
    )
    def _sc_scatter(hp, src_cat, dst_cat, zerosH, out, sidx, didx, rows, acc,
                    isem, gsem, ssem):
        c = lax.axis_index("c")
        s = lax.axis_index("s")
        ebase = c * E + s * EPT
        soff = jnp.full((16,), c * N, jnp.int32)
        pltpu.sync_copy(zerosH.at[pl.ds(s * RPT, RPT)], acc.at[pl.ds(s * RPT, RPT)])
        plsc.subcore_barrier()

        def i_start(j):
            b = j % _NIB
            pltpu.async_copy(src_cat.at[pl.ds(ebase + j * C, C)], sidx.at[b],
                             isem.at[b])
            pltpu.async_copy(dst_cat.at[pl.ds(ebase + j * C, C)], didx.at[b],
                             isem.at[b])

        def i_wait(j):
            b = j % _NIB
            pltpu.make_async_copy(src_cat.at[pl.ds(ebase + j * C, C)],
                                  sidx.at[b], isem.at[b]).wait()
            pltpu.make_async_copy(dst_cat.at[pl.ds(ebase + j * C, C)],
                                  didx.at[b], isem.at[b]).wait()
            for k in range(C // 16):
                sl = pl.ds(k * 16, 16)
                sidx[b, sl] = sidx[b, sl] + soff

        def g_start(j):
            b = j % _NBUF
            pltpu.async_copy(hp.at[sidx.at[j % _NIB]], rows.at[b], gsem.at[b])

        def g_wait(j):
            b = j % _NBUF
            pltpu.make_async_copy(hp.at[sidx.at[j % _NIB]], rows.at[b],
                                  gsem.at[b]).wait()

        def s_start(j):
            b = j % _NBUF
            pltpu.async_copy(rows.at[b], acc.at[didx.at[j % _NIB]],
                             ssem.at[b], add=True)

        def s_wait(j):
            b = j % _NBUF
            pltpu.make_async_copy(rows.at[b], acc.at[didx.at[j % _NIB]],
                                  ssem.at[b]).wait()

        for t in range(_IAH):
            i_start(t)
        for t in range(_GAH):
            i_wait(t)
            g_start(t)

        def body(j, carry):
            @pl.when(j + _IAH < NCHUNK)
            def _():
                i_start(j + _IAH)

            @pl.when(j + _GAH < NCHUNK)
            def _():
                i_wait(j + _GAH)

                @pl.when(j + _GAH >= _NBUF)
                def _():
                    s_wait(j + _GAH - _NBUF)

                g_start(j + _GAH)

            g_wait(j)
            s_start(j)
            return carry

        lax.fori_loop(0, NCHUNK, body, 0)
        for t in range(_NBUF):
            s_wait(NCHUNK - _NBUF + t)
        plsc.subcore_barrier()
        pltpu.sync_copy(acc.at[pl.ds(s * RPT, RPT)],
                        out.at[pl.ds(c * N + s * RPT, RPT)])

    return _sc_scatter


_sc_scatter_h1 = _make_sc_scatter(H1)
_sc_scatter_h2 = _make_sc_scatter(H2)


_BR = 2000
_GB = N // _BR
_NBLK = M // _BR


def _norm_from(dg):
    return lax.rsqrt(dg[:, 0:1] + 1.0)


def _mm_body(x1, x2, w, o):
    i = pl.program_id(0)
    x = jnp.where(i < _GB, x1[...], x2[...])
    o[...] = jnp.dot(x, w[...], preferred_element_type=jnp.float32)


def _scale_body(dg, xw, o):
    o[...] = (xw[...] * _norm_from(dg)).astype(jnp.bfloat16)


def _layer_mm_body(dg, s1, hp, w, o):
    norm = _norm_from(dg)
    f32 = jnp.float32
    h = jax.nn.relu(norm * (s1[...].astype(f32) + hp[...].astype(f32)))
    o[...] = (jnp.dot(h, w[...], preferred_element_type=f32)
              * norm).astype(jnp.bfloat16)


def _finish_ntn_body(dg, s2, hp, wtT, vT, bn, wo, bo, o, scr):
    i = pl.program_id(0)
    norm = _norm_from(dg)
    f32 = jnp.float32
    h = jax.nn.relu(norm * (s2[...].astype(f32) + hp[...].astype(f32)))
    sums = jnp.sum(h, axis=0, keepdims=True)

    @pl.when(i == 0)
    def _():
        scr[...] = jnp.zeros_like(scr)

    g = i // _GB
    scr[pl.ds(g, 1), :] += sums

    @pl.when(i == _NBLK - 1)
    def _():
        g1 = scr[0:1, :] * (1.0 / N)
        g2 = scr[1:2, :] * (1.0 / N)
        cols = []
        for k in range(K):
            tk = jnp.dot(g1, wtT[k], preferred_element_type=f32)
            cols.append(jnp.sum(tk * g2, axis=1, keepdims=True))
        bil = jnp.concatenate(cols, axis=1)
        cat = jnp.concatenate([g1, g2], axis=1)
        lin = jnp.dot(cat, vT[...], preferred_element_type=f32)
        ntn = jnp.tanh(bil + lin + bn[...])
        sc = jnp.sum(wo[...] * ntn)
        o[...] = jnp.full((1, 1), jax.nn.sigmoid(sc + bo[0, 0]), jnp.float32)


def _row_spec(width):
    return pl.BlockSpec((_BR, width), lambda i: (i, 0))


def _full_spec(shape):
    nd = len(shape)
    return pl.BlockSpec(shape, lambda i: (0,) * nd)


def kernel(x1, edge_index1, x2, edge_index2, W1, W2, Wt, V, b_ntn, w_out, b_out):
    f32 = jnp.float32
    bf16 = jnp.bfloat16
    src_cat = jnp.concatenate([edge_index1[0], edge_index2[0]]).astype(jnp.int32)
    dst_cat = jnp.concatenate([edge_index1[1], edge_index2[1]]).astype(jnp.int32)

    ones_rows = jnp.zeros((C, 16), f32).at[:, 0].set(1.0)
    zeros16 = jnp.zeros((N, 16), f32)
    zeros64 = jnp.zeros((N, H1), bf16)
    zeros32 = jnp.zeros((N, H2), bf16)

    degp = _sc_degree(dst_cat, ones_rows, zeros16)

    xw1 = pl.pallas_call(
        _mm_body,
        grid=(_NBLK,),
        in_specs=[pl.BlockSpec((_BR, D), lambda i: (i % _GB, 0)),
                  pl.BlockSpec((_BR, D), lambda i: (i % _GB, 0)),
                  pl.BlockSpec((D, H1), lambda i: (0, 0))],
        out_specs=_row_spec(H1),
        out_shape=jax.ShapeDtypeStruct((M, H1), f32),
    )(x1, x2, W1)
    h1p = pl.pallas_call(
        _scale_body,
        grid=(_NBLK,),
        in_specs=[_row_spec(16), _row_spec(H1)],
        out_specs=_row_spec(H1),
        out_shape=jax.ShapeDtypeStruct((M, H1), bf16),
    )(degp, xw1)

    s1 = _sc_scatter_h1(h1p, src_cat, dst_cat, zeros64)

    h2p = pl.pallas_call(
        _layer_mm_body,
        grid=(_NBLK,),
        in_specs=[_row_spec(16), _row_spec(H1), _row_spec(H1),
                  pl.BlockSpec((H1, H2), lambda i: (0, 0))],
        out_specs=_row_spec(H2),
        out_shape=jax.ShapeDtypeStruct((M, H2), bf16),
    )(degp, s1, h1p, W2)

    s2 = _sc_scatter_h2(h2p, src_cat, dst_cat, zeros32)

    wtT = jnp.transpose(Wt, (2, 0, 1)).astype(f32)
    vT = jnp.transpose(V).astype(f32)
    bn = b_ntn.reshape(1, K).astype(f32)
    wo = w_out.reshape(1, K).astype(f32)
    bo = b_out.reshape(1, 1).astype(f32)
    score = pl.pallas_call(
        _finish_ntn_body,
        grid=(_NBLK,),
        in_specs=[_row_spec(16), _row_spec(H2), _row_spec(H2),
                  _full_spec((K, H2, H2)), _full_spec((2 * H2, K)),
                  _full_spec((1, K)), _full_spec((1, K)), _full_spec((1, 1))],
        out_specs=_full_spec((1, 1)),
        out_shape=jax.ShapeDtypeStruct((1, 1), f32),
        scratch_shapes=[pltpu.VMEM((8, H2), f32)],
    )(degp, s2, h2p, wtT, vT, bn, wo, bo)

    return score.reshape(())


# speedup vs baseline: 1.0055x; 1.0055x over previous
import functools

import jax
import jax.numpy as jnp
from jax import lax
from jax.experimental import pallas as pl
from jax.experimental.pallas import tpu as pltpu
from jax.experimental.pallas import tpu_sc as plsc

N = 10000
E = 320000
D = 128
H1 = 64
H2 = 32
K = 16

M = 2 * N
EPT = E // 16
C = 80
NCHUNK = EPT // C
RPT = N // 16

_mesh = plsc.VectorSubcoreMesh(core_axis_name="c", subcore_axis_name="s")
_sc_params = pltpu.CompilerParams(use_tc_tiling_on_sc=False)


_NIB = 8
_IAH = 4
_SDEP = 4
_NBUF = 4
_GAH = 2


@functools.partial(
    pl.kernel,
    out_type=jax.ShapeDtypeStruct((M, 16), jnp.float32),
    mesh=_mesh,
    compiler_params=_sc_params,
    scratch_types=[
        pltpu.VMEM((_NIB, C), jnp.int32),
        pltpu.VMEM((C, 16), jnp.float32),
        pltpu.VMEM_SHARED((N, 16), jnp.float32),
        pltpu.SemaphoreType.DMA((_NIB,)),
        pltpu.SemaphoreType.DMA,
    ],
)
def _sc_degree(dst_cat, ones_rows, zeros16, out, didx, ones_v, acc, isem, dsem):
    c = lax.axis_index("c")
    s = lax.axis_index("s")
    ebase = c * E + s * EPT
    pltpu.sync_copy(ones_rows, ones_v)
    pltpu.sync_copy(zeros16.at[pl.ds(s * RPT, RPT)], acc.at[pl.ds(s * RPT, RPT)])
    plsc.subcore_barrier()

    def i_start(j):
        b = j % _NIB
        pltpu.async_copy(dst_cat.at[pl.ds(ebase + j * C, C)], didx.at[b],
                         isem.at[b])

    def i_wait(j):
        b = j % _NIB
        pltpu.make_async_copy(dst_cat.at[pl.ds(ebase + j * C, C)], didx.at[b],
                              isem.at[b]).wait()

    def s_drain(_j):
        pltpu.make_async_copy(ones_v, acc.at[didx.at[_j % _NIB]], dsem).wait()

    for t in range(_IAH):
        i_start(t)

    def body(j, carry):
        @pl.when(j >= _SDEP)
        def _():
            s_drain(j - _SDEP)

        @pl.when(j + _IAH < NCHUNK)
        def _():
            i_start(j + _IAH)

        i_wait(j)
        pltpu.async_copy(ones_v, acc.at[didx.at[j % _NIB]], dsem, add=True)
        return carry

    lax.fori_loop(0, NCHUNK, body, 0)
    for t in range(_SDEP):
        s_drain(NCHUNK - _SDEP + t)
    plsc.subcore_barrier()
    pltpu.sync_copy(acc.at[pl.ds(s * RPT, RPT)],
                    out.at[pl.ds(c * N + s * RPT, RPT)])


def _make_sc_scatter(H):
    @functools.partial(
        pl.kernel,
        out_type=jax.ShapeDtypeStruct((M, H), jnp.bfloat16),
        mesh=_mesh,
        compiler_params=_sc_params,
        scratch_types=[
            pltpu.VMEM((_NIB, C), jnp.int32),
            pltpu.VMEM((_NIB, C), jnp.int32),
            pltpu.VMEM((_NBUF, C, H), jnp.bfloat16),
            pltpu.VMEM_SHARED((N, H), jnp.bfloat16),
            pltpu.SemaphoreType.DMA((_NIB,)),
            pltpu.SemaphoreType.DMA((_NBUF,)),
            pltpu.SemaphoreType.DMA((_NBUF,)),
        ],
    )
    def _sc_scatter(hp, src_cat, dst_cat, zerosH, out, sidx, didx, rows, acc,
                    isem, gsem, ssem):
        c = lax.axis_index("c")
        s = lax.axis_index("s")
        ebase = c * E + s * EPT
        soff = jnp.full((16,), c * N, jnp.int32)
        pltpu.sync_copy(zerosH.at[pl.ds(s * RPT, RPT)], acc.at[pl.ds(s * RPT, RPT)])
        plsc.subcore_barrier()

        def i_start(j):
            b = j % _NIB
            pltpu.async_copy(src_cat.at[pl.ds(ebase + j * C, C)], sidx.at[b],
                             isem.at[b])
            pltpu.async_copy(dst_cat.at[pl.ds(ebase + j * C, C)], didx.at[b],
                             isem.at[b])

        def i_wait(j):
            b = j % _NIB
            pltpu.make_async_copy(src_cat.at[pl.ds(ebase + j * C, C)],
                                  sidx.at[b], isem.at[b]).wait()
            pltpu.make_async_copy(dst_cat.at[pl.ds(ebase + j * C, C)],
                                  didx.at[b], isem.at[b]).wait()
            for k in range(C // 16):
                sl = pl.ds(k * 16, 16)
                sidx[b, sl] = sidx[b, sl] + soff

        def g_start(j):
            b = j % _NBUF
            pltpu.async_copy(hp.at[sidx.at[j % _NIB]], rows.at[b], gsem.at[b])

        def g_wait(j):
            b = j % _NBUF
            pltpu.make_async_copy(hp.at[sidx.at[j % _NIB]], rows.at[b],
                                  gsem.at[b]).wait()

        def s_start(j):
            b = j % _NBUF
            pltpu.async_copy(rows.at[b], acc.at[didx.at[j % _NIB]],
                             ssem.at[b], add=True)

        def s_wait(j):
            b = j % _NBUF
            pltpu.make_async_copy(rows.at[b], acc.at[didx.at[j % _NIB]],
                                  ssem.at[b]).wait()

        for t in range(_IAH):
            i_start(t)
        for t in range(_GAH):
            i_wait(t)
            g_start(t)

        def body(j, carry):
            @pl.when(j + _IAH < NCHUNK)
            def _():
                i_start(j + _IAH)

            @pl.when(j + _GAH < NCHUNK)
            def _():
                i_wait(j + _GAH)

                @pl.when(j + _GAH >= _NBUF)
                def _():
                    s_wait(j + _GAH - _NBUF)

                g_start(j + _GAH)

            g_wait(j)
            s_start(j)
            return carry

        lax.fori_loop(0, NCHUNK, body, 0)
        for t in range(_NBUF):
            s_wait(NCHUNK - _NBUF + t)
        plsc.subcore_barrier()
        pltpu.sync_copy(acc.at[pl.ds(s * RPT, RPT)],
                        out.at[pl.ds(c * N + s * RPT, RPT)])

    return _sc_scatter


_sc_scatter_h1 = _make_sc_scatter(H1)
_sc_scatter_h2 = _make_sc_scatter(H2)


_BR = 2000
_GB = N // _BR
_NBLK = M // _BR


def _norm_from(dg):
    return lax.rsqrt(dg[:, 0:1] + 1.0)


def _mm_scale_body(dg, x1, x2, w, o):
    i = pl.program_id(0)
    norm = _norm_from(dg)
    x = jnp.where(i < _GB, x1[...], x2[...])
    o[...] = (jnp.dot(x, w[...], preferred_element_type=jnp.float32)
              * norm).astype(jnp.bfloat16)


def _layer_mm_body(dg, s1, hp, w, o):
    norm = _norm_from(dg)
    f32 = jnp.float32
    h = jax.nn.relu(norm * (s1[...].astype(f32) + hp[...].astype(f32)))
    o[...] = (jnp.dot(h, w[...], preferred_element_type=f32)
              * norm).astype(jnp.bfloat16)


def _finish_ntn_body(dg, s2, hp, wtT, vT, bn, wo, bo, o, scr):
    i = pl.program_id(0)
    norm = _norm_from(dg)
    f32 = jnp.float32
    h = jax.nn.relu(norm * (s2[...].astype(f32) + hp[...].astype(f32)))
    sums = jnp.sum(h, axis=0, keepdims=True)

    @pl.when(i == 0)
    def _():
        scr[...] = jnp.zeros_like(scr)

    g = i // _GB
    scr[pl.ds(g, 1), :] += sums

    @pl.when(i == _NBLK - 1)
    def _():
        g1 = scr[0:1, :] * (1.0 / N)
        g2 = scr[1:2, :] * (1.0 / N)
        cols = []
        for k in range(K):
            tk = jnp.dot(g1, wtT[k], preferred_element_type=f32)
            cols.append(jnp.sum(tk * g2, axis=1, keepdims=True))
        bil = jnp.concatenate(cols, axis=1)
        cat = jnp.concatenate([g1, g2], axis=1)
        lin = jnp.dot(cat, vT[...], preferred_element_type=f32)
        ntn = jnp.tanh(bil + lin + bn[...])
        sc = jnp.sum(wo[...] * ntn)
        o[...] = jnp.full((1, 1), jax.nn.sigmoid(sc + bo[0, 0]), jnp.float32)


def _row_spec(width):
    return pl.BlockSpec((_BR, width), lambda i: (i, 0))


def _full_spec(shape):
    nd = len(shape)
    return pl.BlockSpec(shape, lambda i: (0,) * nd)


def kernel(x1, edge_index1, x2, edge_index2, W1, W2, Wt, V, b_ntn, w_out, b_out):
    f32 = jnp.float32
    bf16 = jnp.bfloat16
    src_cat = jnp.concatenate([edge_index1[0], edge_index2[0]]).astype(jnp.int32)
    dst_cat = jnp.concatenate([edge_index1[1], edge_index2[1]]).astype(jnp.int32)

    ones_rows = jnp.zeros((C, 16), f32).at[:, 0].set(1.0)
    zeros16 = jnp.zeros((N, 16), f32)
    zeros64 = jnp.zeros((N, H1), bf16)
    zeros32 = jnp.zeros((N, H2), bf16)

    degp = _sc_degree(dst_cat, ones_rows, zeros16)

    h1p = pl.pallas_call(
        _mm_scale_body,
        grid=(_NBLK,),
        in_specs=[_row_spec(16),
                  pl.BlockSpec((_BR, D), lambda i: (i % _GB, 0)),
                  pl.BlockSpec((_BR, D), lambda i: (i % _GB, 0)),
                  pl.BlockSpec((D, H1), lambda i: (0, 0))],
        out_specs=_row_spec(H1),
        out_shape=jax.ShapeDtypeStruct((M, H1), bf16),
    )(degp, x1, x2, W1)

    s1 = _sc_scatter_h1(h1p, src_cat, dst_cat, zeros64)

    h2p = pl.pallas_call(
        _layer_mm_body,
        grid=(_NBLK,),
        in_specs=[_row_spec(16), _row_spec(H1), _row_spec(H1),
                  pl.BlockSpec((H1, H2), lambda i: (0, 0))],
        out_specs=_row_spec(H2),
        out_shape=jax.ShapeDtypeStruct((M, H2), bf16),
    )(degp, s1, h1p, W2)

    s2 = _sc_scatter_h2(h2p, src_cat, dst_cat, zeros32)

    wtT = jnp.transpose(Wt, (2, 0, 1)).astype(f32)
    vT = jnp.transpose(V).astype(f32)
    bn = b_ntn.reshape(1, K).astype(f32)
    wo = w_out.reshape(1, K).astype(f32)
    bo = b_out.reshape(1, 1).astype(f32)
    score = pl.pallas_call(
        _finish_ntn_body,
        grid=(_NBLK,),
        in_specs=[_row_spec(16), _row_spec(H2), _row_spec(H2),
                  _full_spec((K, H2, H2)), _full_spec((2 * H2, K)),
                  _full_spec((1, K)), _full_spec((1, K)), _full_spec((1, 1))],
        out_specs=_full_spec((1, 1)),
        out_shape=jax.ShapeDtypeStruct((1, 1), f32),
        scratch_shapes=[pltpu.VMEM((8, H2), f32)],
    )(degp, s2, h2p, wtT, vT, bn, wo, bo)

    return score.reshape(())
